# baseline (speedup 1.0000x reference)
import functools

import jax
import jax.numpy as jnp
from jax import lax
from jax.experimental import pallas as pl
from jax.experimental.pallas import tpu as pltpu
from jax.experimental.pallas import tpu_sc as plsc

_N = 4096
_D = 768
_H = 512
_E = 64
_B = 128
_G = 128
_P = _G * _B
_TB = 512
_NA = 1024
_RB = 2048
_MB = 2048
_NW = 32
_TPW = _N // _NW


_D2 = _D // 2


def _pack_bf16(a):
    ar = a.astype(jnp.bfloat16).astype(jnp.float32)
    bits = lax.bitcast_convert_type(ar, jnp.uint32)
    hi = bits[:, :_D2] & jnp.uint32(0xFFFF0000)
    lo = jnp.right_shift(bits[:, _D2:], jnp.uint32(16))
    return lax.bitcast_convert_type(hi | lo, jnp.int32)


def _unpack_bf16(p):
    pu = lax.bitcast_convert_type(p, jnp.uint32)
    hi = lax.bitcast_convert_type(pu & jnp.uint32(0xFFFF0000), jnp.float32)
    lo = lax.bitcast_convert_type(jnp.left_shift(pu, jnp.uint32(16)),
                                  jnp.float32)
    return jnp.concatenate([hi, lo], axis=1)


def _router_body(x_ref, rw_ref, rb_ref, ids0_ref, ids1_ref, w0_ref, w1_ref,
                 xbf_ref):
    xb = x_ref[...]
    xbf_ref[...] = _pack_bf16(xb)
    logits = lax.dot_general(xb, rw_ref[...], (((1,), (1,)), ((), ())),
                             preferred_element_type=jnp.float32)
    logits = logits + rb_ref[...][None, :]

    iota_e = lax.broadcasted_iota(jnp.int32, (_RB, _E), 1)
    m1 = jnp.max(logits, axis=1, keepdims=True)
    a1 = jnp.min(jnp.where(logits == m1, iota_e, _E), axis=1)
    masked = jnp.where(iota_e == a1[:, None], -jnp.inf, logits)
    m2 = jnp.max(masked, axis=1, keepdims=True)
    a2 = jnp.min(jnp.where(masked == m2, iota_e, _E), axis=1)

    w0 = jax.nn.sigmoid(m1[:, 0] - m2[:, 0])
    ids0_ref[...] = a1.astype(jnp.int32)
    ids1_ref[...] = a2.astype(jnp.int32)
    w0_ref[...] = w0
    w1_ref[...] = 1.0 - w0


def _router(x_flat, router_w, expert_bias):
    grid = (_N // _RB,)
    return pl.pallas_call(
        _router_body,
        grid=grid,
        in_specs=[
            pl.BlockSpec((_RB, _D), lambda i: (i, 0)),
            pl.BlockSpec((_E, _D), lambda i: (0, 0)),
            pl.BlockSpec((_E,), lambda i: (0,)),
        ],
        out_specs=[
            pl.BlockSpec((_RB,), lambda i: (i,)),
            pl.BlockSpec((_RB,), lambda i: (i,)),
            pl.BlockSpec((_RB,), lambda i: (i,)),
            pl.BlockSpec((_RB,), lambda i: (i,)),
            pl.BlockSpec((_RB, _D2), lambda i: (i, 0)),
        ],
        out_shape=[
            jax.ShapeDtypeStruct((_N,), jnp.int32),
            jax.ShapeDtypeStruct((_N,), jnp.int32),
            jax.ShapeDtypeStruct((_N,), jnp.float32),
            jax.ShapeDtypeStruct((_N,), jnp.float32),
            jax.ShapeDtypeStruct((_N, _D2), jnp.int32),
        ],
    )(x_flat, router_w, expert_bias)


def _shared_body(x_ref, sw1_ref, sw2_ref, sw3_ref, sh_ref):
    xbh = x_ref[...].astype(jnp.bfloat16)
    acc = jnp.zeros((_TB, _D), jnp.float32)
    for i in range(sw1_ref.shape[0]):
        h1 = lax.dot_general(xbh, sw1_ref[i].astype(jnp.bfloat16),
                             (((1,), (1,)), ((), ())),
                             preferred_element_type=jnp.float32)
        h3 = lax.dot_general(xbh, sw3_ref[i].astype(jnp.bfloat16),
                             (((1,), (1,)), ((), ())),
                             preferred_element_type=jnp.float32)
        h = (jax.nn.silu(h1) * h3).astype(jnp.bfloat16)
        acc = acc + lax.dot_general(h, sw2_ref[i].astype(jnp.bfloat16),
                                    (((1,), (1,)), ((), ())),
                                    preferred_element_type=jnp.float32)
    sh_ref[...] = _pack_bf16(acc)


def _shared_ffn(x_part, sw1, sw2, sw3):
    nshared = sw1.shape[0]
    ntok = x_part.shape[0]
    grid = (ntok // _TB,)
    return pl.pallas_call(
        _shared_body,
        grid=grid,
        in_specs=[
            pl.BlockSpec((_TB, _D), lambda i: (i, 0)),
            pl.BlockSpec((nshared, _H, _D), lambda i: (0, 0, 0)),
            pl.BlockSpec((nshared, _D, _H), lambda i: (0, 0, 0)),
            pl.BlockSpec((nshared, _H, _D), lambda i: (0, 0, 0)),
        ],
        out_specs=pl.BlockSpec((_TB, _D2), lambda i: (i, 0)),
        out_shape=jax.ShapeDtypeStruct((ntok, _D2), jnp.int32),
    )(x_part, sw1, sw2, sw3)


def _meta_body(ids0_ref, ids1_ref, pos0_ref, pos1_ref, meta_ref,
               acc_ref, rank_ref, tri_ref):
    b = pl.program_id(0)
    nhalf = _N // _MB
    nrb = (2 * _N) // _MB

    @pl.when(b == 0)
    def _():
        acc_ref[...] = jnp.zeros((_E,), jnp.float32)
        r_iota = lax.broadcasted_iota(jnp.int32, (_MB, _MB), 0)
        c_iota = lax.broadcasted_iota(jnp.int32, (_MB, _MB), 1)
        tri_ref[...] = (c_iota < r_iota).astype(jnp.bfloat16)

    @pl.when(b < nrb)
    def _():
        off0 = jnp.where(b < nhalf, b * _MB, 0)
        off1 = jnp.where(b < nhalf, 0, (b - nhalf) * _MB)
        ia = ids0_ref[pl.ds(off0, _MB)]
        ib = ids1_ref[pl.ds(off1, _MB)]
        ids = jnp.where(b < nhalf, ia, ib)

        iota_e = lax.broadcasted_iota(jnp.int32, (_MB, _E), 1)
        oh_bf = (ids[:, None] == iota_e).astype(jnp.bfloat16)
        oh = oh_bf.astype(jnp.float32)
        excl = lax.dot_general(tri_ref[...], oh_bf, (((1,), (0,)), ((), ())),
                               preferred_element_type=jnp.float32)
        rank_rows = jnp.sum(oh * (excl + acc_ref[...][None, :]), axis=1)
        rank_ref[pl.ds(b * _MB, _MB)] = rank_rows
        acc_ref[...] = acc_ref[...] + jnp.sum(oh, axis=0)

    @pl.when(b == nrb)
    def _():
        counts = acc_ref[...]
        padded = jnp.floor((counts + (_B - 1.0)) / _B) * _B
        re_iota = lax.broadcasted_iota(jnp.int32, (_E, _E), 0)
        ce_iota = lax.broadcasted_iota(jnp.int32, (_E, _E), 1)
        tri64 = (ce_iota < re_iota).astype(jnp.float32)
        offs = lax.dot_general(tri64, padded[:, None],
                               (((1,), (0,)), ((), ())),
                               preferred_element_type=jnp.float32)[:, 0]
        cum_incl = offs + padded
        nbf = jnp.sum(padded) / _B
        gidx = lax.broadcasted_iota(jnp.int32, (_G, _E), 0).astype(
            jnp.float32) * _B
        be_raw = jnp.sum((gidx >= cum_incl[None, :]).astype(jnp.float32),
                         axis=1)
        e_vec = lax.broadcasted_iota(jnp.int32, (1, _E), 1).astype(
            jnp.float32)[0]
        last = jnp.max(e_vec * (counts > 0).astype(jnp.float32))
        be = jnp.minimum(be_raw, last)
        meta_ref[0:1, :] = be.astype(jnp.int32)[None, :]
        meta_ref[1:2, :] = (jnp.zeros((_G,), jnp.float32)
                            + nbf).astype(jnp.int32)[None, :]

        iota_e0 = lax.broadcasted_iota(jnp.int32, (_N, _E), 1)
        oh0 = (ids0_ref[...][:, None] == iota_e0).astype(jnp.float32)
        oh1 = (ids1_ref[...][:, None] == iota_e0).astype(jnp.float32)
        p0 = jnp.sum(oh0 * offs[None, :], axis=1) + rank_ref[pl.ds(0, _N)]
        p1 = jnp.sum(oh1 * offs[None, :], axis=1) + rank_ref[pl.ds(_N, _N)]
        pos0_ref[...] = p0.astype(jnp.int32)
        pos1_ref[...] = p1.astype(jnp.int32)


def _meta(ids0, ids1):
    nrb = (2 * _N) // _MB
    return pl.pallas_call(
        _meta_body,
        grid=(nrb + 1,),
        in_specs=[
            pl.BlockSpec((_N,), lambda b: (0,)),
            pl.BlockSpec((_N,), lambda b: (0,)),
        ],
        out_specs=[
            pl.BlockSpec((_N,), lambda b: (0,)),
            pl.BlockSpec((_N,), lambda b: (0,)),
            pl.BlockSpec((2, _G), lambda b: (0, 0)),
        ],
        out_shape=[
            jax.ShapeDtypeStruct((_N,), jnp.int32),
            jax.ShapeDtypeStruct((_N,), jnp.int32),
            jax.ShapeDtypeStruct((2, _G), jnp.int32),
        ],
        scratch_shapes=[
            pltpu.VMEM((_E,), jnp.float32),
            pltpu.VMEM((2 * _N,), jnp.float32),
            pltpu.VMEM((_MB, _MB), jnp.bfloat16),
        ],
    )(ids0, ids1)


def _dispatch_body(x_hbm, pos0_hbm, pos1_hbm, xs_hbm,
                   idx0_v, idx1_v, rows_v, sem0, sem1):
    wid = lax.axis_index("s") * 2 + lax.axis_index("c")
    base = wid * _TPW
    pltpu.sync_copy(pos0_hbm.at[pl.ds(base, _TPW)], idx0_v)
    pltpu.sync_copy(pos1_hbm.at[pl.ds(base, _TPW)], idx1_v)
    pltpu.sync_copy(x_hbm.at[pl.ds(base, _TPW)], rows_v)
    cp0 = pltpu.async_copy(rows_v, xs_hbm.at[idx0_v], sem0)
    cp1 = pltpu.async_copy(rows_v, xs_hbm.at[idx1_v], sem1)
    cp0.wait()
    cp1.wait()


def _dispatch(x_bf, pos0, pos1):
    mesh = plsc.VectorSubcoreMesh(core_axis_name="c", subcore_axis_name="s")
    f = pl.kernel(
        _dispatch_body,
        out_type=jax.ShapeDtypeStruct((_P, _D2), jnp.int32),
        mesh=mesh,
        scratch_types=[
            pltpu.VMEM((_TPW,), jnp.int32),
            pltpu.VMEM((_TPW,), jnp.int32),
            pltpu.VMEM((_TPW, _D2), jnp.int32),
            pltpu.SemaphoreType.DMA,
            pltpu.SemaphoreType.DMA,
        ],
    )
    return f(x_bf, pos0, pos1)


def _ffn_body(be_ref, nb_ref, xs_ref, w1_ref, w2_ref, w3_ref, ys_ref):
    g = pl.program_id(0)
    @pl.when(jnp.logical_or(g < nb_ref[0], g == _G - 1))
    def _():
        xb = _unpack_bf16(xs_ref[...]).astype(jnp.bfloat16)
        h1 = lax.dot_general(xb, w1_ref[0].astype(jnp.bfloat16),
                             (((1,), (1,)), ((), ())),
                             preferred_element_type=jnp.float32)
        h3 = lax.dot_general(xb, w3_ref[0].astype(jnp.bfloat16),
                             (((1,), (1,)), ((), ())),
                             preferred_element_type=jnp.float32)
        h = (jax.nn.silu(h1) * h3).astype(jnp.bfloat16)
        y = lax.dot_general(h, w2_ref[0].astype(jnp.bfloat16),
                            (((1,), (1,)), ((), ())),
                            preferred_element_type=jnp.float32)
        ys_ref[...] = _pack_bf16(y)


def _grouped_ffn(be, nb, xs, ew1, ew2, ew3):
    grid_spec = pltpu.PrefetchScalarGridSpec(
        num_scalar_prefetch=2,
        grid=(_G,),
        in_specs=[
            pl.BlockSpec((_B, _D2),
                         lambda g, be, nb: (jnp.minimum(g, nb[0] - 1), 0)),
            pl.BlockSpec((1, _H, _D), lambda g, be, nb: (be[g], 0, 0)),
            pl.BlockSpec((1, _D, _H), lambda g, be, nb: (be[g], 0, 0)),
            pl.BlockSpec((1, _H, _D), lambda g, be, nb: (be[g], 0, 0)),
        ],
        out_specs=pl.BlockSpec((_B, _D2),
                               lambda g, be, nb: (jnp.minimum(g, nb[0] - 1),
                                                  0)),
    )
    return pl.pallas_call(
        _ffn_body,
        grid_spec=grid_spec,
        out_shape=jax.ShapeDtypeStruct((_P, _D2), jnp.int32),
    )(be, nb, xs, ew1, ew2, ew3)


def _gatherback_body(ys_hbm, pos0_hbm, pos1_hbm, y0_hbm, y1_hbm,
                     idx_v, rows_v, sem):
    wid = lax.axis_index("s") * 2 + lax.axis_index("c")
    base = wid * _TPW
    pltpu.sync_copy(pos0_hbm.at[pl.ds(base, _TPW)], idx_v)
    pltpu.async_copy(ys_hbm.at[idx_v], rows_v, sem).wait()
    pltpu.sync_copy(rows_v, y0_hbm.at[pl.ds(base, _TPW)])
    pltpu.sync_copy(pos1_hbm.at[pl.ds(base, _TPW)], idx_v)
    pltpu.async_copy(ys_hbm.at[idx_v], rows_v, sem).wait()
    pltpu.sync_copy(rows_v, y1_hbm.at[pl.ds(base, _TPW)])


def _gatherback(ys, pos0, pos1):
    mesh = plsc.VectorSubcoreMesh(core_axis_name="c", subcore_axis_name="s")
    f = pl.kernel(
        _gatherback_body,
        out_type=(
            jax.ShapeDtypeStruct((_N, _D2), jnp.int32),
            jax.ShapeDtypeStruct((_N, _D2), jnp.int32),
        ),
        mesh=mesh,
        scratch_types=[
            pltpu.VMEM((_TPW,), jnp.int32),
            pltpu.VMEM((_TPW, _D2), jnp.int32),
            pltpu.SemaphoreType.DMA,
        ],
    )
    return f(ys, pos0, pos1)


def _combine_body(sha_ref, shb_ref, y0_ref, y1_ref, w0_ref, w1_ref, out_ref):
    i = pl.program_id(0)
    nba = _NA // _TB
    sh = jnp.where(i < nba, sha_ref[...], shb_ref[...])
    out_ref[...] = (_unpack_bf16(sh)
                    + w0_ref[...][:, None] * _unpack_bf16(y0_ref[...])
                    + w1_ref[...][:, None] * _unpack_bf16(y1_ref[...]))


def _combine(sh_a, sh_b, y0, y1, w0, w1):
    grid = (_N // _TB,)
    nba = _NA // _TB
    nbb = (_N - _NA) // _TB
    return pl.pallas_call(
        _combine_body,
        grid=grid,
        in_specs=[
            pl.BlockSpec((_TB, _D2),
                         lambda i: (jnp.minimum(i, nba - 1), 0)),
            pl.BlockSpec((_TB, _D2),
                         lambda i: (jnp.clip(i - nba, 0, nbb - 1), 0)),
            pl.BlockSpec((_TB, _D2), lambda i: (i, 0)),
            pl.BlockSpec((_TB, _D2), lambda i: (i, 0)),
            pl.BlockSpec((_TB,), lambda i: (i,)),
            pl.BlockSpec((_TB,), lambda i: (i,)),
        ],
        out_specs=pl.BlockSpec((_TB, _D), lambda i: (i, 0)),
        out_shape=jax.ShapeDtypeStruct((_N, _D), jnp.float32),
    )(sh_a, sh_b, y0, y1, w0, w1)


def kernel(x, router_w, expert_bias, shared_w1, shared_w2, shared_w3,
           expert_w1, expert_w2, expert_w3):
    bs, seq, d = x.shape
    x_flat = x.reshape(-1, d)

    ids0, ids1, w0, w1, x_bf = _router(x_flat, router_w, expert_bias)

    pos0, pos1, meta = _meta(ids0, ids1)
    be = meta[0]
    nb = meta[1, 0:1]

    xs = _dispatch(x_bf, pos0, pos1)
    sh_a = _shared_ffn(x_flat[:_NA], shared_w1, shared_w2, shared_w3)
    ys = _grouped_ffn(be, nb, xs, expert_w1, expert_w2, expert_w3)
    sh_b = _shared_ffn(x_flat[_NA:], shared_w1, shared_w2, shared_w3)
    y0, y1 = _gatherback(ys, pos0, pos1)
    out = _combine(sh_a, sh_b, y0, y1, w0, w1)
    return out.reshape(bs, seq, d)


# speedup vs baseline: 1.0862x; 1.0862x over previous
import functools

import jax
import jax.numpy as jnp
from jax import lax
from jax.experimental import pallas as pl
from jax.experimental.pallas import tpu as pltpu
from jax.experimental.pallas import tpu_sc as plsc

_N = 4096
_D = 768
_H = 512
_E = 64
_B = 128
_G = 128
_P = _G * _B
_TB = 512
_NA = 1024
_RB = 2048
_MB = 2048
_NW = 32
_TPW = _N // _NW


_D2 = _D // 2


def _pack_bf16(a):
    ar = a.astype(jnp.bfloat16).astype(jnp.float32)
    bits = lax.bitcast_convert_type(ar, jnp.uint32)
    hi = bits[:, :_D2] & jnp.uint32(0xFFFF0000)
    lo = jnp.right_shift(bits[:, _D2:], jnp.uint32(16))
    return lax.bitcast_convert_type(hi | lo, jnp.int32)


def _unpack_bf16(p):
    pu = lax.bitcast_convert_type(p, jnp.uint32)
    hi = lax.bitcast_convert_type(pu & jnp.uint32(0xFFFF0000), jnp.float32)
    lo = lax.bitcast_convert_type(jnp.left_shift(pu, jnp.uint32(16)),
                                  jnp.float32)
    return jnp.concatenate([hi, lo], axis=1)


def _router_body(x_ref, rw_ref, rb_ref, ids0_ref, ids1_ref, w0_ref, w1_ref,
                 xbf_ref):
    xb = x_ref[...]
    xbf_ref[...] = _pack_bf16(xb)
    logits = lax.dot_general(xb, rw_ref[...], (((1,), (1,)), ((), ())),
                             preferred_element_type=jnp.float32)
    logits = logits + rb_ref[...][None, :]

    iota_e = lax.broadcasted_iota(jnp.int32, (_RB, _E), 1)
    m1 = jnp.max(logits, axis=1, keepdims=True)
    a1 = jnp.min(jnp.where(logits == m1, iota_e, _E), axis=1)
    masked = jnp.where(iota_e == a1[:, None], -jnp.inf, logits)
    m2 = jnp.max(masked, axis=1, keepdims=True)
    a2 = jnp.min(jnp.where(masked == m2, iota_e, _E), axis=1)

    w0 = jax.nn.sigmoid(m1[:, 0] - m2[:, 0])
    ids0_ref[...] = a1.astype(jnp.int32)
    ids1_ref[...] = a2.astype(jnp.int32)
    w0_ref[...] = w0
    w1_ref[...] = 1.0 - w0


def _router(x_flat, router_w, expert_bias):
    grid = (_N // _RB,)
    return pl.pallas_call(
        _router_body,
        grid=grid,
        in_specs=[
            pl.BlockSpec((_RB, _D), lambda i: (i, 0)),
            pl.BlockSpec((_E, _D), lambda i: (0, 0)),
            pl.BlockSpec((_E,), lambda i: (0,)),
        ],
        out_specs=[
            pl.BlockSpec((_RB,), lambda i: (i,)),
            pl.BlockSpec((_RB,), lambda i: (i,)),
            pl.BlockSpec((_RB,), lambda i: (i,)),
            pl.BlockSpec((_RB,), lambda i: (i,)),
            pl.BlockSpec((_RB, _D2), lambda i: (i, 0)),
        ],
        out_shape=[
            jax.ShapeDtypeStruct((_N,), jnp.int32),
            jax.ShapeDtypeStruct((_N,), jnp.int32),
            jax.ShapeDtypeStruct((_N,), jnp.float32),
            jax.ShapeDtypeStruct((_N,), jnp.float32),
            jax.ShapeDtypeStruct((_N, _D2), jnp.int32),
        ],
    )(x_flat, router_w, expert_bias)


def _shared_body(x_ref, sw1_ref, sw2_ref, sw3_ref, sh_ref):
    xbh = x_ref[...].astype(jnp.bfloat16)
    acc = jnp.zeros((_TB, _D), jnp.float32)
    for i in range(sw1_ref.shape[0]):
        h1 = lax.dot_general(xbh, sw1_ref[i].astype(jnp.bfloat16),
                             (((1,), (1,)), ((), ())),
                             preferred_element_type=jnp.float32)
        h3 = lax.dot_general(xbh, sw3_ref[i].astype(jnp.bfloat16),
                             (((1,), (1,)), ((), ())),
                             preferred_element_type=jnp.float32)
        h = (jax.nn.silu(h1) * h3).astype(jnp.bfloat16)
        acc = acc + lax.dot_general(h, sw2_ref[i].astype(jnp.bfloat16),
                                    (((1,), (1,)), ((), ())),
                                    preferred_element_type=jnp.float32)
    sh_ref[...] = _pack_bf16(acc)


def _shared_ffn(x_part, sw1, sw2, sw3):
    nshared = sw1.shape[0]
    ntok = x_part.shape[0]
    grid = (ntok // _TB,)
    return pl.pallas_call(
        _shared_body,
        grid=grid,
        in_specs=[
            pl.BlockSpec((_TB, _D), lambda i: (i, 0)),
            pl.BlockSpec((nshared, _H, _D), lambda i: (0, 0, 0)),
            pl.BlockSpec((nshared, _D, _H), lambda i: (0, 0, 0)),
            pl.BlockSpec((nshared, _H, _D), lambda i: (0, 0, 0)),
        ],
        out_specs=pl.BlockSpec((_TB, _D2), lambda i: (i, 0)),
        out_shape=jax.ShapeDtypeStruct((ntok, _D2), jnp.int32),
    )(x_part, sw1, sw2, sw3)


def _meta_body(ids0_ref, ids1_ref, pos0_ref, pos1_ref, meta_ref,
               acc_ref, rank_ref, tri_ref):
    b = pl.program_id(0)
    nhalf = _N // _MB
    nrb = (2 * _N) // _MB

    @pl.when(b == 0)
    def _():
        acc_ref[...] = jnp.zeros((_E,), jnp.float32)
        r_iota = lax.broadcasted_iota(jnp.int32, (_MB, _MB), 0)
        c_iota = lax.broadcasted_iota(jnp.int32, (_MB, _MB), 1)
        tri_ref[...] = (c_iota < r_iota).astype(jnp.bfloat16)

    @pl.when(b < nrb)
    def _():
        off0 = jnp.where(b < nhalf, b * _MB, 0)
        off1 = jnp.where(b < nhalf, 0, (b - nhalf) * _MB)
        ia = ids0_ref[pl.ds(off0, _MB)]
        ib = ids1_ref[pl.ds(off1, _MB)]
        ids = jnp.where(b < nhalf, ia, ib)

        iota_e = lax.broadcasted_iota(jnp.int32, (_MB, _E), 1)
        oh_bf = (ids[:, None] == iota_e).astype(jnp.bfloat16)
        oh = oh_bf.astype(jnp.float32)
        excl = lax.dot_general(tri_ref[...], oh_bf, (((1,), (0,)), ((), ())),
                               preferred_element_type=jnp.float32)
        rank_rows = jnp.sum(oh * (excl + acc_ref[...][None, :]), axis=1)
        rank_ref[pl.ds(b * _MB, _MB)] = rank_rows
        acc_ref[...] = acc_ref[...] + jnp.sum(oh, axis=0)

    @pl.when(b == nrb)
    def _():
        counts = acc_ref[...]
        padded = jnp.floor((counts + (_B - 1.0)) / _B) * _B
        re_iota = lax.broadcasted_iota(jnp.int32, (_E, _E), 0)
        ce_iota = lax.broadcasted_iota(jnp.int32, (_E, _E), 1)
        tri64 = (ce_iota < re_iota).astype(jnp.float32)
        offs = lax.dot_general(tri64, padded[:, None],
                               (((1,), (0,)), ((), ())),
                               preferred_element_type=jnp.float32)[:, 0]
        cum_incl = offs + padded
        nbf = jnp.sum(padded) / _B
        gidx = lax.broadcasted_iota(jnp.int32, (_G, _E), 0).astype(
            jnp.float32) * _B
        be_raw = jnp.sum((gidx >= cum_incl[None, :]).astype(jnp.float32),
                         axis=1)
        e_vec = lax.broadcasted_iota(jnp.int32, (1, _E), 1).astype(
            jnp.float32)[0]
        last = jnp.max(e_vec * (counts > 0).astype(jnp.float32))
        be = jnp.minimum(be_raw, last)
        meta_ref[0:1, :] = be.astype(jnp.int32)[None, :]
        meta_ref[1:2, :] = (jnp.zeros((_G,), jnp.float32)
                            + nbf).astype(jnp.int32)[None, :]

        iota_e0 = lax.broadcasted_iota(jnp.int32, (_N, _E), 1)
        oh0 = (ids0_ref[...][:, None] == iota_e0).astype(jnp.float32)
        oh1 = (ids1_ref[...][:, None] == iota_e0).astype(jnp.float32)
        p0 = jnp.sum(oh0 * offs[None, :], axis=1) + rank_ref[pl.ds(0, _N)]
        p1 = jnp.sum(oh1 * offs[None, :], axis=1) + rank_ref[pl.ds(_N, _N)]
        pos0_ref[...] = p0.astype(jnp.int32)
        pos1_ref[...] = p1.astype(jnp.int32)


def _meta(ids0, ids1):
    nrb = (2 * _N) // _MB
    return pl.pallas_call(
        _meta_body,
        grid=(nrb + 1,),
        in_specs=[
            pl.BlockSpec((_N,), lambda b: (0,)),
            pl.BlockSpec((_N,), lambda b: (0,)),
        ],
        out_specs=[
            pl.BlockSpec((_N,), lambda b: (0,)),
            pl.BlockSpec((_N,), lambda b: (0,)),
            pl.BlockSpec((2, _G), lambda b: (0, 0)),
        ],
        out_shape=[
            jax.ShapeDtypeStruct((_N,), jnp.int32),
            jax.ShapeDtypeStruct((_N,), jnp.int32),
            jax.ShapeDtypeStruct((2, _G), jnp.int32),
        ],
        scratch_shapes=[
            pltpu.VMEM((_E,), jnp.float32),
            pltpu.VMEM((2 * _N,), jnp.float32),
            pltpu.VMEM((_MB, _MB), jnp.bfloat16),
        ],
    )(ids0, ids1)


def _dispatch_body(x_hbm, pos0_hbm, pos1_hbm, xs_hbm,
                   idx0_v, idx1_v, rows_v, sem0, sem1):
    wid = lax.axis_index("s") * 2 + lax.axis_index("c")
    base = wid * _TPW
    pltpu.sync_copy(pos0_hbm.at[pl.ds(base, _TPW)], idx0_v)
    pltpu.sync_copy(pos1_hbm.at[pl.ds(base, _TPW)], idx1_v)
    pltpu.sync_copy(x_hbm.at[pl.ds(base, _TPW)], rows_v)
    cp0 = pltpu.async_copy(rows_v, xs_hbm.at[idx0_v], sem0)
    cp1 = pltpu.async_copy(rows_v, xs_hbm.at[idx1_v], sem1)
    cp0.wait()
    cp1.wait()


def _dispatch(x_bf, pos0, pos1):
    mesh = plsc.VectorSubcoreMesh(core_axis_name="c", subcore_axis_name="s")
    f = pl.kernel(
        _dispatch_body,
        out_type=jax.ShapeDtypeStruct((_P, _D2), jnp.int32),
        mesh=mesh,
        scratch_types=[
            pltpu.VMEM((_TPW,), jnp.int32),
            pltpu.VMEM((_TPW,), jnp.int32),
            pltpu.VMEM((_TPW, _D2), jnp.int32),
            pltpu.SemaphoreType.DMA,
            pltpu.SemaphoreType.DMA,
        ],
    )
    return f(x_bf, pos0, pos1)


def _ffn_body(be_ref, nb_ref, xs_ref, w1_ref, w2_ref, w3_ref, ys_ref):
    g = pl.program_id(0)
    @pl.when(jnp.logical_or(g < nb_ref[0], g == _G - 1))
    def _():
        xb = _unpack_bf16(xs_ref[...]).astype(jnp.bfloat16)
        h1 = lax.dot_general(xb, w1_ref[0].astype(jnp.bfloat16),
                             (((1,), (1,)), ((), ())),
                             preferred_element_type=jnp.float32)
        h3 = lax.dot_general(xb, w3_ref[0].astype(jnp.bfloat16),
                             (((1,), (1,)), ((), ())),
                             preferred_element_type=jnp.float32)
        h = (jax.nn.silu(h1) * h3).astype(jnp.bfloat16)
        y = lax.dot_general(h, w2_ref[0].astype(jnp.bfloat16),
                            (((1,), (1,)), ((), ())),
                            preferred_element_type=jnp.float32)
        ys_ref[...] = _pack_bf16(y)


def _grouped_ffn(be, nb, xs, ew1, ew2, ew3):
    grid_spec = pltpu.PrefetchScalarGridSpec(
        num_scalar_prefetch=2,
        grid=(_G,),
        in_specs=[
            pl.BlockSpec((_B, _D2),
                         lambda g, be, nb: (jnp.minimum(g, nb[0] - 1), 0)),
            pl.BlockSpec((1, _H, _D), lambda g, be, nb: (be[g], 0, 0)),
            pl.BlockSpec((1, _D, _H), lambda g, be, nb: (be[g], 0, 0)),
            pl.BlockSpec((1, _H, _D), lambda g, be, nb: (be[g], 0, 0)),
        ],
        out_specs=pl.BlockSpec((_B, _D2),
                               lambda g, be, nb: (jnp.minimum(g, nb[0] - 1),
                                                  0)),
    )
    return pl.pallas_call(
        _ffn_body,
        grid_spec=grid_spec,
        out_shape=jax.ShapeDtypeStruct((_P, _D2), jnp.int32),
    )(be, nb, xs, ew1, ew2, ew3)


def _gatherback_body(ys_hbm, pos0_hbm, pos1_hbm, y0_hbm, y1_hbm,
                     idx_v, rows_v, sem):
    wid = lax.axis_index("s") * 2 + lax.axis_index("c")
    base = wid * _TPW
    pltpu.sync_copy(pos0_hbm.at[pl.ds(base, _TPW)], idx_v)
    pltpu.async_copy(ys_hbm.at[idx_v], rows_v, sem).wait()
    pltpu.sync_copy(rows_v, y0_hbm.at[pl.ds(base, _TPW)])
    pltpu.sync_copy(pos1_hbm.at[pl.ds(base, _TPW)], idx_v)
    pltpu.async_copy(ys_hbm.at[idx_v], rows_v, sem).wait()
    pltpu.sync_copy(rows_v, y1_hbm.at[pl.ds(base, _TPW)])


def _gatherback(ys, pos0, pos1):
    mesh = plsc.VectorSubcoreMesh(core_axis_name="c", subcore_axis_name="s")
    f = pl.kernel(
        _gatherback_body,
        out_type=(
            jax.ShapeDtypeStruct((_N, _D2), jnp.int32),
            jax.ShapeDtypeStruct((_N, _D2), jnp.int32),
        ),
        mesh=mesh,
        scratch_types=[
            pltpu.VMEM((_TPW,), jnp.int32),
            pltpu.VMEM((_TPW, _D2), jnp.int32),
            pltpu.SemaphoreType.DMA,
        ],
    )
    return f(ys, pos0, pos1)


def _combine_body(sh_ref, y0_ref, y1_ref, w0_ref, w1_ref, out_ref):
    out_ref[...] = (_unpack_bf16(sh_ref[...])
                    + w0_ref[...][:, None] * _unpack_bf16(y0_ref[...])
                    + w1_ref[...][:, None] * _unpack_bf16(y1_ref[...]))


def _combine(sh, y0, y1, w0, w1):
    grid = (_N // _TB,)
    return pl.pallas_call(
        _combine_body,
        grid=grid,
        in_specs=[
            pl.BlockSpec((_TB, _D2), lambda i: (i, 0)),
            pl.BlockSpec((_TB, _D2), lambda i: (i, 0)),
            pl.BlockSpec((_TB, _D2), lambda i: (i, 0)),
            pl.BlockSpec((_TB,), lambda i: (i,)),
            pl.BlockSpec((_TB,), lambda i: (i,)),
        ],
        out_specs=pl.BlockSpec((_TB, _D), lambda i: (i, 0)),
        out_shape=jax.ShapeDtypeStruct((_N, _D), jnp.float32),
    )(sh, y0, y1, w0, w1)


def kernel(x, router_w, expert_bias, shared_w1, shared_w2, shared_w3,
           expert_w1, expert_w2, expert_w3):
    bs, seq, d = x.shape
    x_flat = x.reshape(-1, d)

    ids0, ids1, w0, w1, x_bf = _router(x_flat, router_w, expert_bias)

    pos0, pos1, meta = _meta(ids0, ids1)
    be = meta[0]
    nb = meta[1, 0:1]

    xs = _dispatch(x_bf, pos0, pos1)
    ys = _grouped_ffn(be, nb, xs, expert_w1, expert_w2, expert_w3)
    shared = _shared_ffn(x_flat, shared_w1, shared_w2, shared_w3)
    y0, y1 = _gatherback(ys, pos0, pos1)
    out = _combine(shared, y0, y1, w0, w1)
    return out.reshape(bs, seq, d)
